# Initial kernel scaffold; baseline (speedup 1.0000x reference)
import functools

import jax
import jax.numpy as jnp
from jax import lax
from jax.experimental import pallas as pl
from jax.experimental.pallas import tpu as pltpu
from jax.experimental.pallas import tpu_sc as plsc

NC, NS = 2, 16
NW = NC * NS
KB = 128
N, E, D = 10000, 320000, 128
EB = (E // NW) // KB * KB
NEXTRA = (E - EB * NW) // KB
ROWS_PT = 624

_mesh = plsc.VectorSubcoreMesh(core_axis_name="c", subcore_axis_name="s")


def _sc_degree(col):
    @functools.partial(
        pl.kernel,
        out_type=jax.ShapeDtypeStruct((NC, N, 16), jnp.float32),
        mesh=_mesh,
        scratch_types=[
            pltpu.VMEM((KB,), jnp.int32),
            pltpu.VMEM((KB, 16), jnp.float32),
            pltpu.VMEM((ROWS_PT, 16), jnp.float32),
            pltpu.VMEM_SHARED((N, 16), jnp.float32),
        ],
    )
    def deg_kernel(col_hbm, degp_hbm, cbuf, ones, zbuf, degs):
        cid = lax.axis_index("c")
        sid = lax.axis_index("s")
        wid = cid * NS + sid

        one_v = jnp.full((16,), 1.0, dtype=jnp.float32)
        zero_v = jnp.zeros((16,), dtype=jnp.float32)

        @pl.loop(0, KB)
        def _(i):
            ones[i, :] = one_v

        @pl.loop(0, ROWS_PT)
        def _(i):
            zbuf[i, :] = zero_v

        pltpu.sync_copy(zbuf, degs.at[pl.ds(sid * ROWS_PT, ROWS_PT)])

        @pl.when(sid == 0)
        def _():
            pltpu.sync_copy(zbuf.at[pl.ds(0, 16)],
                            degs.at[pl.ds(NS * ROWS_PT, 16)])

        plsc.subcore_barrier()

        @pl.loop(0, EB // KB)
        def _(b):
            base = wid * EB + b * KB
            pltpu.sync_copy(col_hbm.at[pl.ds(base, KB)], cbuf)
            pltpu.sync_copy(ones, degs.at[cbuf], add=True)

        @pl.when(wid < NEXTRA)
        def _():
            base = NW * EB + wid * KB
            pltpu.sync_copy(col_hbm.at[pl.ds(base, KB)], cbuf)
            pltpu.sync_copy(ones, degs.at[cbuf], add=True)

        plsc.subcore_barrier()

        out = degp_hbm.at[cid]
        pltpu.sync_copy(degs.at[pl.ds(sid * ROWS_PT, ROWS_PT)],
                        out.at[pl.ds(sid * ROWS_PT, ROWS_PT)])

        @pl.when(sid == 0)
        def _():
            pltpu.sync_copy(degs.at[pl.ds(NS * ROWS_PT, 16)],
                            out.at[pl.ds(NS * ROWS_PT, 16)])

    return deg_kernel(col)


def _sc_aggregate(edge_index, g):
    @functools.partial(
        pl.kernel,
        out_type=jax.ShapeDtypeStruct((NC, N, D), jnp.float32),
        mesh=_mesh,
        scratch_types=[
            pltpu.VMEM((KB,), jnp.int32),
            pltpu.VMEM((KB,), jnp.int32),
            pltpu.VMEM((KB, D), jnp.float32),
            pltpu.VMEM_SHARED((N, D), jnp.float32),
            pltpu.SemaphoreType.DMA,
        ],
    )
    def agg_kernel(ei_hbm, g_hbm, raw_hbm, rbuf, cbuf, gbuf, aggs, sem):
        cid = lax.axis_index("c")
        sid = lax.axis_index("s")
        wid = cid * NS + sid

        zero_v = jnp.zeros((16,), dtype=jnp.float32)

        @pl.loop(0, KB)
        def _(i):
            @pl.loop(0, D // 16)
            def _(j):
                gbuf[i, pl.ds(j * 16, 16)] = zero_v

        row0 = sid * ROWS_PT
        for k in range(ROWS_PT // KB):
            pltpu.sync_copy(gbuf, aggs.at[pl.ds(row0 + k * KB, KB)])
        rem = ROWS_PT % KB
        pltpu.sync_copy(gbuf.at[pl.ds(0, rem)],
                        aggs.at[pl.ds(row0 + (ROWS_PT // KB) * KB, rem)])

        @pl.when(sid == 0)
        def _():
            pltpu.sync_copy(gbuf.at[pl.ds(0, 16)],
                            aggs.at[pl.ds(NS * ROWS_PT, 16)])

        plsc.subcore_barrier()

        def do_batch(base):
            pltpu.sync_copy(ei_hbm.at[0, pl.ds(base, KB)], rbuf)
            pltpu.sync_copy(ei_hbm.at[1, pl.ds(base, KB)], cbuf)
            pltpu.async_copy(g_hbm.at[rbuf], gbuf, sem).wait()
            pltpu.sync_copy(gbuf, aggs.at[cbuf], add=True)

        @pl.loop(0, EB // KB)
        def _(b):
            do_batch(wid * EB + b * KB)

        @pl.when(wid < NEXTRA)
        def _():
            do_batch(NW * EB + wid * KB)

        plsc.subcore_barrier()

        out = raw_hbm.at[cid]
        pltpu.sync_copy(aggs.at[pl.ds(sid * ROWS_PT, ROWS_PT)],
                        out.at[pl.ds(sid * ROWS_PT, ROWS_PT)])

        @pl.when(sid == 0)
        def _():
            pltpu.sync_copy(aggs.at[pl.ds(NS * ROWS_PT, 16)],
                            out.at[pl.ds(NS * ROWS_PT, 16)])

    return agg_kernel(edge_index, g)


_GRID = 5
_BM = N // _GRID


def _mm0(x, W0, b0):
    def body(x_ref, w_ref, b_ref, o_ref):
        o_ref[...] = jnp.dot(x_ref[...], w_ref[...],
                             preferred_element_type=jnp.float32) + b_ref[...]

    return pl.pallas_call(
        body,
        grid=(_GRID,),
        in_specs=[
            pl.BlockSpec((_BM, D), lambda i: (i, 0)),
            pl.BlockSpec((D, D), lambda i: (0, 0)),
            pl.BlockSpec((1, D), lambda i: (0, 0)),
        ],
        out_specs=pl.BlockSpec((_BM, D), lambda i: (i, 0)),
        out_shape=jax.ShapeDtypeStruct((N, D), jnp.float32),
    )(x, W0, b0.reshape(1, D))


def _dis_of(d_ref):
    deg = d_ref[0, :, 0:1] + d_ref[1, :, 0:1]
    return jnp.where(deg > 0, lax.rsqrt(deg), 0.0)


def _scale(h, degp):
    def body(h_ref, d_ref, o_ref):
        o_ref[...] = h_ref[...] * _dis_of(d_ref)

    return pl.pallas_call(
        body,
        grid=(_GRID,),
        in_specs=[
            pl.BlockSpec((_BM, D), lambda i: (i, 0)),
            pl.BlockSpec((NC, _BM, 16), lambda i: (0, i, 0)),
        ],
        out_specs=pl.BlockSpec((_BM, D), lambda i: (i, 0)),
        out_shape=jax.ShapeDtypeStruct((N, D), jnp.float32),
    )(h, degp)


def _finale(rawp, degp, h, W1, b1):
    def body(r_ref, d_ref, h_ref, w_ref, b_ref, o_ref):
        agg = r_ref[0] + r_ref[1]
        a = 0.5 * (_dis_of(d_ref) * agg) + 0.5 * h_ref[...]
        nrm = jnp.sqrt(jnp.sum(a * a, axis=1, keepdims=True))
        a = a / jnp.maximum(nrm, 1e-12)
        a = jnp.maximum(a, 0.0)
        o_ref[...] = jnp.dot(a, w_ref[...],
                             preferred_element_type=jnp.float32) + b_ref[...]

    return pl.pallas_call(
        body,
        grid=(_GRID,),
        in_specs=[
            pl.BlockSpec((NC, _BM, D), lambda i: (0, i, 0)),
            pl.BlockSpec((NC, _BM, 16), lambda i: (0, i, 0)),
            pl.BlockSpec((_BM, D), lambda i: (i, 0)),
            pl.BlockSpec((D, D), lambda i: (0, 0)),
            pl.BlockSpec((1, D), lambda i: (0, 0)),
        ],
        out_specs=pl.BlockSpec((_BM, D), lambda i: (i, 0)),
        out_shape=jax.ShapeDtypeStruct((N, D), jnp.float32),
    )(rawp, degp, h, W1, b1.reshape(1, D))


def kernel(x, edge_index, W0, b0, W1, b1):
    degp = _sc_degree(edge_index[1])
    h = _mm0(x, W0, b0)
    g = _scale(h, degp)
    rawp = _sc_aggregate(edge_index, g)
    return _finale(rawp, degp, h, W1, b1)


# speedup vs baseline: 15.6318x; 15.6318x over previous
import functools

import jax
import jax.numpy as jnp
from jax import lax
from jax.experimental import pallas as pl
from jax.experimental.pallas import tpu as pltpu
from jax.experimental.pallas import tpu_sc as plsc

NC, NS = 2, 16
NW = NC * NS
KB = 128
N, E, D = 10000, 320000, 128
EB = (E // NW) // KB * KB
NEXTRA = (E - EB * NW) // KB
ROWS_PT = 624

_mesh = plsc.VectorSubcoreMesh(core_axis_name="c", subcore_axis_name="s")


def _sc_degree(col):
    CH = 80
    EPT = E // NW

    @functools.partial(
        pl.kernel,
        out_type=jax.ShapeDtypeStruct((NC * N,), jnp.float32),
        mesh=_mesh,
        scratch_types=[
            pltpu.VMEM((CH,), jnp.int32),
            pltpu.VMEM((CH,), jnp.float32),
            pltpu.VMEM((N,), jnp.float32),
            pltpu.VMEM_SHARED((N,), jnp.float32),
        ],
    )
    def deg_kernel(col_hbm, degp_hbm, cbuf, ones, zbuf, degs):
        cid = lax.axis_index("c")
        sid = lax.axis_index("s")
        wid = cid * NS + sid

        one_v = jnp.full((16,), 1.0, dtype=jnp.float32)
        zero_v = jnp.zeros((16,), dtype=jnp.float32)

        @pl.loop(0, CH // 16)
        def _(i):
            ones[pl.ds(i * 16, 16)] = one_v

        @pl.loop(0, N // 16)
        def _(i):
            zbuf[pl.ds(i * 16, 16)] = zero_v

        pltpu.sync_copy(zbuf, degs)
        plsc.subcore_barrier()

        @pl.loop(0, EPT // CH)
        def _(b):
            base = wid * EPT + b * CH
            pltpu.sync_copy(col_hbm.at[pl.ds(base, CH)], cbuf)
            pltpu.sync_copy(ones, degs.at[cbuf], add=True)

        plsc.subcore_barrier()

        pltpu.sync_copy(degs, zbuf)
        pltpu.sync_copy(zbuf, degp_hbm.at[pl.ds(cid * N, N)])

    return deg_kernel(col)


def _sc_aggregate(row, col, g):
    CH = 80
    EPT = E // NW
    RPT = 624

    @functools.partial(
        pl.kernel,
        out_type=jax.ShapeDtypeStruct((NC * N, D), jnp.float32),
        mesh=_mesh,
        scratch_types=[
            pltpu.VMEM((CH,), jnp.int32),
            pltpu.VMEM((CH,), jnp.int32),
            pltpu.VMEM((CH, D), jnp.float32),
            pltpu.VMEM_SHARED((N, D), jnp.float32),
            pltpu.SemaphoreType.DMA,
        ],
    )
    def agg_kernel(row_hbm, col_hbm, g_hbm, raw_hbm, rbuf, cbuf, gbuf, aggs,
                   sem):
        cid = lax.axis_index("c")
        sid = lax.axis_index("s")
        wid = cid * NS + sid

        zero_v = jnp.zeros((16,), dtype=jnp.float32)

        @pl.loop(0, CH)
        def _(i):
            @pl.loop(0, D // 16)
            def _(j):
                gbuf[i, pl.ds(j * 16, 16)] = zero_v

        row0 = sid * RPT
        for k in range(RPT // CH):
            pltpu.sync_copy(gbuf, aggs.at[pl.ds(row0 + k * CH, CH)])
        pltpu.sync_copy(gbuf.at[pl.ds(0, RPT % CH)],
                        aggs.at[pl.ds(row0 + (RPT // CH) * CH, RPT % CH)])
        pltpu.sync_copy(gbuf.at[pl.ds(0, 16)], aggs.at[pl.ds(NS * RPT, 16)])

        plsc.subcore_barrier()

        @pl.loop(0, EPT // CH)
        def _(b):
            base = wid * EPT + b * CH
            pltpu.sync_copy(row_hbm.at[pl.ds(base, CH)], rbuf)
            pltpu.sync_copy(col_hbm.at[pl.ds(base, CH)], cbuf)
            pltpu.async_copy(g_hbm.at[rbuf], gbuf, sem).wait()
            pltpu.sync_copy(gbuf, aggs.at[cbuf], add=True)

        plsc.subcore_barrier()

        out0 = cid * N
        for k in range(RPT // CH):
            r = row0 + k * CH
            pltpu.sync_copy(aggs.at[pl.ds(r, CH)], gbuf)
            pltpu.sync_copy(gbuf, raw_hbm.at[pl.ds(out0 + r, CH)])
        r = row0 + (RPT // CH) * CH
        pltpu.sync_copy(aggs.at[pl.ds(r, RPT % CH)], gbuf.at[pl.ds(0, RPT % CH)])
        pltpu.sync_copy(gbuf.at[pl.ds(0, RPT % CH)],
                        raw_hbm.at[pl.ds(out0 + r, RPT % CH)])
        pltpu.sync_copy(aggs.at[pl.ds(NS * RPT, 16)], gbuf.at[pl.ds(0, 16)])
        pltpu.sync_copy(gbuf.at[pl.ds(0, 16)],
                        raw_hbm.at[pl.ds(out0 + NS * RPT, 16)])

    return agg_kernel(row, col, g)


_GRID = 5
_BM = N // _GRID


def _mm0(x, W0, b0):
    def body(x_ref, w_ref, b_ref, o_ref):
        o_ref[...] = jnp.dot(x_ref[...], w_ref[...],
                             preferred_element_type=jnp.float32) + b_ref[...]

    return pl.pallas_call(
        body,
        grid=(_GRID,),
        in_specs=[
            pl.BlockSpec((_BM, D), lambda i: (i, 0)),
            pl.BlockSpec((D, D), lambda i: (0, 0)),
            pl.BlockSpec((1, D), lambda i: (0, 0)),
        ],
        out_specs=pl.BlockSpec((_BM, D), lambda i: (i, 0)),
        out_shape=jax.ShapeDtypeStruct((N, D), jnp.float32),
    )(x, W0, b0.reshape(1, D))


def _dis_of(d_ref):
    deg = d_ref[0, :, 0:1] + d_ref[1, :, 0:1]
    return jnp.where(deg > 0, lax.rsqrt(deg), 0.0)


def _scale(h, degp):
    def body(h_ref, d_ref, o_ref):
        o_ref[...] = h_ref[...] * _dis_of(d_ref)

    return pl.pallas_call(
        body,
        grid=(_GRID,),
        in_specs=[
            pl.BlockSpec((_BM, D), lambda i: (i, 0)),
            pl.BlockSpec((NC, _BM, 16), lambda i: (0, i, 0)),
        ],
        out_specs=pl.BlockSpec((_BM, D), lambda i: (i, 0)),
        out_shape=jax.ShapeDtypeStruct((N, D), jnp.float32),
    )(h, degp)


def _finale(rawp, degp, h, W1, b1):
    def body(r_ref, d_ref, h_ref, w_ref, b_ref, o_ref):
        agg = r_ref[0] + r_ref[1]
        a = 0.5 * (_dis_of(d_ref) * agg) + 0.5 * h_ref[...]
        nrm = jnp.sqrt(jnp.sum(a * a, axis=1, keepdims=True))
        a = a / jnp.maximum(nrm, 1e-12)
        a = jnp.maximum(a, 0.0)
        o_ref[...] = jnp.dot(a, w_ref[...],
                             preferred_element_type=jnp.float32) + b_ref[...]

    return pl.pallas_call(
        body,
        grid=(_GRID,),
        in_specs=[
            pl.BlockSpec((NC, _BM, D), lambda i: (0, i, 0)),
            pl.BlockSpec((NC, _BM, 16), lambda i: (0, i, 0)),
            pl.BlockSpec((_BM, D), lambda i: (i, 0)),
            pl.BlockSpec((D, D), lambda i: (0, 0)),
            pl.BlockSpec((1, D), lambda i: (0, 0)),
        ],
        out_specs=pl.BlockSpec((_BM, D), lambda i: (i, 0)),
        out_shape=jax.ShapeDtypeStruct((N, D), jnp.float32),
    )(rawp, degp, h, W1, b1.reshape(1, D))


def kernel(x, edge_index, W0, b0, W1, b1):
    row, col = edge_index[0], edge_index[1]
    degf = _sc_degree(col)
    degp = jnp.broadcast_to(degf.reshape(NC, N, 1), (NC, N, 16))
    h = _mm0(x, W0, b0)
    g = _scale(h, degp)
    rawp = _sc_aggregate(row, col, g).reshape(NC, N, D)
    return _finale(rawp, degp, h, W1, b1)


# speedup vs baseline: 35.1545x; 2.2489x over previous
import functools

import jax
import jax.numpy as jnp
from jax import lax
from jax.experimental import pallas as pl
from jax.experimental.pallas import tpu as pltpu
from jax.experimental.pallas import tpu_sc as plsc

NC, NS = 2, 16
NW = NC * NS
N, E, D = 10000, 320000, 128
NB, CH = 125, 80

_mesh = plsc.VectorSubcoreMesh(core_axis_name="c", subcore_axis_name="s")


def _sc_degree(col3):
    @functools.partial(
        pl.kernel,
        out_type=jax.ShapeDtypeStruct((NC * N,), jnp.float32),
        mesh=_mesh,
        scratch_types=[
            pltpu.VMEM((NB, CH), jnp.int32),
            pltpu.VMEM((CH,), jnp.float32),
            pltpu.VMEM((N,), jnp.float32),
            pltpu.VMEM_SHARED((N,), jnp.float32),
            pltpu.SemaphoreType.DMA,
        ],
    )
    def deg_kernel(col_hbm, degp_hbm, cbuf2, ones, zbuf, degs, sem):
        cid = lax.axis_index("c")
        sid = lax.axis_index("s")
        wid = cid * NS + sid

        one_v = jnp.full((16,), 1.0, dtype=jnp.float32)
        zero_v = jnp.zeros((16,), dtype=jnp.float32)

        @pl.loop(0, CH // 16)
        def _(i):
            ones[pl.ds(i * 16, 16)] = one_v

        @pl.loop(0, N // 16)
        def _(i):
            zbuf[pl.ds(i * 16, 16)] = zero_v

        pltpu.sync_copy(zbuf, degs)
        plsc.subcore_barrier()

        pltpu.sync_copy(col_hbm.at[wid], cbuf2)

        @pl.loop(0, NB // 5)
        def _(p):
            for j in range(5):
                pltpu.make_async_copy(
                    ones, degs.at[cbuf2.at[5 * p + j]], sem).start(add=True)
            for j in range(5):
                pltpu.make_async_copy(
                    ones, degs.at[cbuf2.at[5 * p + j]], sem).wait()

        plsc.subcore_barrier()

        pltpu.sync_copy(degs, zbuf)
        pltpu.sync_copy(zbuf, degp_hbm.at[pl.ds(cid * N, N)])

    return deg_kernel(col3)


def _sc_aggregate(row1, col3, g):
    RPT = 624

    @functools.partial(
        pl.kernel,
        out_type=jax.ShapeDtypeStruct((NC * N, D), jnp.float32),
        mesh=_mesh,
        scratch_types=[
            pltpu.VMEM((NB * CH,), jnp.int32),
            pltpu.VMEM((NB, CH), jnp.int32),
            pltpu.VMEM((CH, D), jnp.float32),
            pltpu.VMEM((CH, D), jnp.float32),
            pltpu.VMEM_SHARED((N, D), jnp.float32),
            pltpu.SemaphoreType.DMA,
            pltpu.SemaphoreType.DMA,
        ],
    )
    def agg_kernel(row_hbm, col_hbm, g_hbm, raw_hbm, rbig, cbuf2,
                   gbuf0, gbuf1, aggs, sem0, sem1):
        cid = lax.axis_index("c")
        sid = lax.axis_index("s")
        wid = cid * NS + sid

        zero_v = jnp.zeros((16,), dtype=jnp.float32)

        @pl.loop(0, CH)
        def _(i):
            @pl.loop(0, D // 16)
            def _(j):
                gbuf0[i, pl.ds(j * 16, 16)] = zero_v

        row0 = sid * RPT
        for k in range(7):
            pltpu.sync_copy(gbuf0, aggs.at[pl.ds(row0 + k * 80, 80)])
        pltpu.sync_copy(gbuf0.at[pl.ds(0, 64)],
                        aggs.at[pl.ds(row0 + 560, 64)])
        pltpu.sync_copy(gbuf0.at[pl.ds(0, 16)], aggs.at[pl.ds(NS * RPT, 16)])

        plsc.subcore_barrier()

        pltpu.sync_copy(row_hbm.at[pl.ds(wid * NB * CH, NB * CH)], rbig)
        pltpu.sync_copy(col_hbm.at[wid], cbuf2)

        def gather(b, buf, sem):
            idx = rbig.at[pl.ds(b * CH, CH)]
            return pltpu.make_async_copy(g_hbm.at[idx], buf, sem)

        def scat(b, buf):
            pltpu.sync_copy(buf, aggs.at[cbuf2.at[b]], add=True)

        gather(0, gbuf0, sem0).start()

        @pl.loop(0, (NB - 1) // 2)
        def _(p):
            b0 = 2 * p
            gather(b0 + 1, gbuf1, sem1).start()
            gather(b0, gbuf0, sem0).wait()
            scat(b0, gbuf0)
            gather(b0 + 2, gbuf0, sem0).start()
            gather(b0 + 1, gbuf1, sem1).wait()
            scat(b0 + 1, gbuf1)

        gather(NB - 1, gbuf0, sem0).wait()
        scat(NB - 1, gbuf0)

        plsc.subcore_barrier()

        out0 = cid * N
        for k in range(7):
            r = row0 + k * 80
            pltpu.sync_copy(aggs.at[pl.ds(r, 80)], gbuf0)
            pltpu.sync_copy(gbuf0, raw_hbm.at[pl.ds(out0 + r, 80)])
        pltpu.sync_copy(aggs.at[pl.ds(row0 + 560, 64)], gbuf0.at[pl.ds(0, 64)])
        pltpu.sync_copy(gbuf0.at[pl.ds(0, 64)],
                        raw_hbm.at[pl.ds(out0 + row0 + 560, 64)])
        pltpu.sync_copy(aggs.at[pl.ds(NS * RPT, 16)], gbuf0.at[pl.ds(0, 16)])
        pltpu.sync_copy(gbuf0.at[pl.ds(0, 16)],
                        raw_hbm.at[pl.ds(out0 + NS * RPT, 16)])

    return agg_kernel(row1, col3, g)


_GRID = 5
_BM = N // _GRID


def _mm0(x, W0, b0):
    def body(x_ref, w_ref, b_ref, o_ref):
        o_ref[...] = jnp.dot(x_ref[...], w_ref[...],
                             preferred_element_type=jnp.float32) + b_ref[...]

    return pl.pallas_call(
        body,
        grid=(_GRID,),
        in_specs=[
            pl.BlockSpec((_BM, D), lambda i: (i, 0)),
            pl.BlockSpec((D, D), lambda i: (0, 0)),
            pl.BlockSpec((1, D), lambda i: (0, 0)),
        ],
        out_specs=pl.BlockSpec((_BM, D), lambda i: (i, 0)),
        out_shape=jax.ShapeDtypeStruct((N, D), jnp.float32),
    )(x, W0, b0.reshape(1, D))


def _dis_of(d_ref):
    deg = d_ref[0, :, 0:1] + d_ref[1, :, 0:1]
    return jnp.where(deg > 0, lax.rsqrt(deg), 0.0)


def _scale(h, degp):
    def body(h_ref, d_ref, o_ref):
        o_ref[...] = h_ref[...] * _dis_of(d_ref)

    return pl.pallas_call(
        body,
        grid=(_GRID,),
        in_specs=[
            pl.BlockSpec((_BM, D), lambda i: (i, 0)),
            pl.BlockSpec((NC, _BM, 16), lambda i: (0, i, 0)),
        ],
        out_specs=pl.BlockSpec((_BM, D), lambda i: (i, 0)),
        out_shape=jax.ShapeDtypeStruct((N, D), jnp.float32),
    )(h, degp)


def _finale(rawp, degp, h, W1, b1):
    def body(r_ref, d_ref, h_ref, w_ref, b_ref, o_ref):
        agg = r_ref[0] + r_ref[1]
        a = 0.5 * (_dis_of(d_ref) * agg) + 0.5 * h_ref[...]
        nrm = jnp.sqrt(jnp.sum(a * a, axis=1, keepdims=True))
        a = a / jnp.maximum(nrm, 1e-12)
        a = jnp.maximum(a, 0.0)
        o_ref[...] = jnp.dot(a, w_ref[...],
                             preferred_element_type=jnp.float32) + b_ref[...]

    return pl.pallas_call(
        body,
        grid=(_GRID,),
        in_specs=[
            pl.BlockSpec((NC, _BM, D), lambda i: (0, i, 0)),
            pl.BlockSpec((NC, _BM, 16), lambda i: (0, i, 0)),
            pl.BlockSpec((_BM, D), lambda i: (i, 0)),
            pl.BlockSpec((D, D), lambda i: (0, 0)),
            pl.BlockSpec((1, D), lambda i: (0, 0)),
        ],
        out_specs=pl.BlockSpec((_BM, D), lambda i: (i, 0)),
        out_shape=jax.ShapeDtypeStruct((N, D), jnp.float32),
    )(rawp, degp, h, W1, b1.reshape(1, D))


def kernel(x, edge_index, W0, b0, W1, b1):
    row1 = edge_index[0]
    col3 = edge_index[1].reshape(NW, NB, CH)
    degf = _sc_degree(col3)
    degp = jnp.broadcast_to(degf.reshape(NC, N, 1), (NC, N, 16))
    h = _mm0(x, W0, b0)
    g = _scale(h, degp)
    rawp = _sc_aggregate(row1, col3, g).reshape(NC, N, D)
    return _finale(rawp, degp, h, W1, b1)
